# baseline (speedup 1.0000x reference)
import functools

import jax
import jax.numpy as jnp
from jax import lax
from jax.experimental import pallas as pl
from jax.experimental.pallas import tpu as pltpu
from jax.experimental.pallas import tpu_sc as plsc

_D = 64
_DP = 128
_B = 4096
_F = 26
_NC, _NS = 2, 16
_NW = _NC * _NS
_BW = _B // _NW
_NB = 8
_NG = _BW // _NB
_RING = 4

_mesh = plsc.VectorSubcoreMesh(core_axis_name="c", subcore_axis_name="s")


@functools.partial(
    pl.kernel,
    mesh=_mesh,
    out_type=jax.ShapeDtypeStruct((_B, _F, _D), jnp.float32),
    scratch_types=[
        pltpu.VMEM((_BW, _F), jnp.int32),
        pltpu.VMEM((_RING, _F, _DP), jnp.float32),
        pltpu.VMEM((2, _NB, _F, _D), jnp.float32),
        pltpu.SemaphoreType.DMA,
        pltpu.SemaphoreType.DMA,
    ],
    compiler_params=pltpu.CompilerParams(use_tc_tiling_on_sc=True),
)
def _gather_rows(table_hbm, idx_hbm, out_hbm, idx_v, gbuf, cbuf, sem, wsem):
    wid = lax.axis_index("s") * _NC + lax.axis_index("c")
    pltpu.sync_copy(idx_hbm.at[pl.ds(wid * _BW, _BW)], idx_v)

    def gather(i, slot):
        return pltpu.make_async_copy(
            table_hbm.at[idx_v.at[i]], gbuf.at[slot], sem
        )

    def compact(slot, kb, wb):
        def two_rows(r2, carry):
            for dr in range(2):
                for k in range(_D // 16):
                    cbuf[wb, kb, r2 * 2 + dr, pl.ds(k * 16, 16)] = gbuf[
                        slot, r2 * 2 + dr, pl.ds(k * 16, 16)
                    ]
            return carry

        lax.fori_loop(0, _F // 2, two_rows, 0)

    def write(ch, wb):
        return pltpu.make_async_copy(
            cbuf.at[wb], out_hbm.at[pl.ds(wid * _BW + ch * _NB, _NB)], wsem
        )

    for p in range(_RING):
        gather(p, p).start()

    def loop_body(ch, carry):
        wb = lax.rem(ch, 2)
        for kb in range(_NB):
            i = ch * _NB + kb
            slot = kb % _RING
            gather(i, slot).wait()
            compact(slot, kb, wb)
            gather(i + _RING, slot).start()
        write(ch, wb).start()
        return carry

    lax.fori_loop(0, 2, loop_body, 0)

    def loop_body2(ch, carry):
        write(ch - 2, lax.rem(ch, 2)).wait()
        return loop_body(ch, carry)

    lax.fori_loop(2, _NG - 1, loop_body2, 0)

    ch = _NG - 1
    wb = ch % 2
    write(ch - 2, wb).wait()
    for kb in range(_NB):
        i = ch * _NB + kb
        slot = kb % _RING
        gather(i, slot).wait()
        compact(slot, kb, wb)
        if i + _RING < _BW:
            gather(i + _RING, slot).start()
    write(ch, wb).start()
    write(_NG - 2, (_NG - 2) % 2).wait()
    write(_NG - 1, wb).wait()


@jax.jit
def kernel(x, embed):
    padded = jnp.concatenate(
        [embed, jnp.zeros((embed.shape[0], _DP - _D), jnp.float32)], axis=1
    )
    return _gather_rows(padded, x.astype(jnp.int32))


# speedup vs baseline: 1.0525x; 1.0525x over previous
import functools

import jax
import jax.numpy as jnp
from jax import lax
from jax.experimental import pallas as pl
from jax.experimental.pallas import tpu as pltpu
from jax.experimental.pallas import tpu_sc as plsc

_D = 64
_DP = 128
_B = 4096
_F = 26
_NC, _NS = 2, 16
_NW = _NC * _NS
_BW = _B // _NW
_NB = 8
_NG = _BW // _NB
_RING = 4

_mesh = plsc.VectorSubcoreMesh(core_axis_name="c", subcore_axis_name="s")


@functools.partial(
    pl.kernel,
    mesh=_mesh,
    out_type=jax.ShapeDtypeStruct((_B, _F, _D), jnp.float32),
    scratch_types=[
        pltpu.VMEM((_BW, _F), jnp.int32),
        pltpu.VMEM((_RING, _F, _DP), jnp.float32),
        pltpu.VMEM((_NB, _F, _D), jnp.float32),
        pltpu.SemaphoreType.DMA,
    ],
    compiler_params=pltpu.CompilerParams(use_tc_tiling_on_sc=True),
)
def _gather_rows(table_hbm, idx_hbm, out_hbm, idx_v, gbuf, cbuf, sem):
    wid = lax.axis_index("s") * _NC + lax.axis_index("c")
    pltpu.sync_copy(idx_hbm.at[pl.ds(wid * _BW, _BW)], idx_v)

    def gather(i, slot):
        return pltpu.make_async_copy(
            table_hbm.at[idx_v.at[i]], gbuf.at[slot], sem
        )

    def compact(slot, kb):
        def two_rows(r2, carry):
            for dr in range(2):
                for k in range(_D // 16):
                    cbuf[kb, r2 * 2 + dr, pl.ds(k * 16, 16)] = gbuf[
                        slot, r2 * 2 + dr, pl.ds(k * 16, 16)
                    ]
            return carry

        lax.fori_loop(0, _F // 2, two_rows, 0)

    for p in range(_RING):
        gather(p, p).start()

    def loop_body(ch, carry):
        for kb in range(_NB):
            i = ch * _NB + kb
            slot = kb % _RING
            gather(i, slot).wait()
            compact(slot, kb)
            gather(i + _RING, slot).start()
        pltpu.sync_copy(cbuf, out_hbm.at[pl.ds(wid * _BW + ch * _NB, _NB)])
        return carry

    lax.fori_loop(0, _NG - 1, loop_body, 0)

    ch = _NG - 1
    for kb in range(_NB):
        i = ch * _NB + kb
        slot = kb % _RING
        gather(i, slot).wait()
        compact(slot, kb)
        if i + _RING < _BW:
            gather(i + _RING, slot).start()
    pltpu.sync_copy(cbuf, out_hbm.at[pl.ds(wid * _BW + ch * _NB, _NB)])


@jax.jit
def kernel(x, embed):
    padded = jnp.concatenate(
        [embed, jnp.zeros((embed.shape[0], _DP - _D), jnp.float32)], axis=1
    )
    return _gather_rows(padded, x.astype(jnp.int32))


# speedup vs baseline: 1.0851x; 1.0309x over previous
import functools

import jax
import jax.numpy as jnp
from jax import lax
from jax.experimental import pallas as pl
from jax.experimental.pallas import tpu as pltpu
from jax.experimental.pallas import tpu_sc as plsc

_D = 64
_DP = 128
_B = 4096
_F = 26
_NC, _NS = 2, 16
_NW = _NC * _NS
_BW = _B // _NW
_NB = 16
_NG = _BW // _NB
_RING = 8

_mesh = plsc.VectorSubcoreMesh(core_axis_name="c", subcore_axis_name="s")


@functools.partial(
    pl.kernel,
    mesh=_mesh,
    out_type=jax.ShapeDtypeStruct((_B, _F, _D), jnp.float32),
    scratch_types=[
        pltpu.VMEM((_BW, _F), jnp.int32),
        pltpu.VMEM((_RING, _F, _DP), jnp.float32),
        pltpu.VMEM((_NB, _F, _D), jnp.float32),
        pltpu.SemaphoreType.DMA,
    ],
    compiler_params=pltpu.CompilerParams(use_tc_tiling_on_sc=True),
)
def _gather_rows(table_hbm, idx_hbm, out_hbm, idx_v, gbuf, cbuf, sem):
    wid = lax.axis_index("s") * _NC + lax.axis_index("c")
    pltpu.sync_copy(idx_hbm.at[pl.ds(wid * _BW, _BW)], idx_v)

    def gather(i, slot):
        return pltpu.make_async_copy(
            table_hbm.at[idx_v.at[i]], gbuf.at[slot], sem
        )

    def compact(slot, kb):
        def two_rows(r2, carry):
            for dr in range(2):
                for k in range(_D // 16):
                    cbuf[kb, r2 * 2 + dr, pl.ds(k * 16, 16)] = gbuf[
                        slot, r2 * 2 + dr, pl.ds(k * 16, 16)
                    ]
            return carry

        lax.fori_loop(0, _F // 2, two_rows, 0)

    for p in range(_RING):
        gather(p, p).start()

    def loop_body(ch, carry):
        for kb in range(_NB):
            i = ch * _NB + kb
            slot = kb % _RING
            gather(i, slot).wait()
            compact(slot, kb)
            gather(i + _RING, slot).start()
        pltpu.sync_copy(cbuf, out_hbm.at[pl.ds(wid * _BW + ch * _NB, _NB)])
        return carry

    lax.fori_loop(0, _NG - 1, loop_body, 0)

    ch = _NG - 1
    for kb in range(_NB):
        i = ch * _NB + kb
        slot = kb % _RING
        gather(i, slot).wait()
        compact(slot, kb)
        if i + _RING < _BW:
            gather(i + _RING, slot).start()
    pltpu.sync_copy(cbuf, out_hbm.at[pl.ds(wid * _BW + ch * _NB, _NB)])


@jax.jit
def kernel(x, embed):
    padded = jnp.concatenate(
        [embed, jnp.zeros((embed.shape[0], _DP - _D), jnp.float32)], axis=1
    )
    return _gather_rows(padded, x.astype(jnp.int32))
